# baseline (speedup 1.0000x reference)
import functools

import jax
import jax.numpy as jnp
from jax import lax
from jax.experimental import pallas as pl
from jax.experimental.pallas import tpu as pltpu
from jax.experimental.pallas import tpu_sc as plsc

N = 2_000_000
NUM_WORKERS = 32
LANES = 16
UNROLL = 8
STEP = LANES * UNROLL
NSUB = 8
SUB = 7_808
CHUNK = NSUB * SUB
TAIL = N - NUM_WORKERS * CHUNK
TABLE_LEN = 119

_mesh = plsc.VectorSubcoreMesh(core_axis_name="c", subcore_axis_name="s")


@functools.partial(
    pl.kernel,
    out_type=jax.ShapeDtypeStruct((N,), jnp.int32),
    mesh=_mesh,
    compiler_params=pltpu.CompilerParams(needs_layout_passes=False),
    scratch_types=[
        pltpu.VMEM((TABLE_LEN,), jnp.int32),
        pltpu.VMEM((SUB,), jnp.int32),
        pltpu.VMEM((SUB,), jnp.int32),
        pltpu.VMEM((SUB,), jnp.int32),
        pltpu.VMEM((SUB,), jnp.int32),
        pltpu.VMEM((TAIL,), jnp.int32),
        pltpu.SemaphoreType.DMA,
        pltpu.SemaphoreType.DMA,
        pltpu.SemaphoreType.DMA,
        pltpu.SemaphoreType.DMA,
    ],
)
def _translate(z_hbm, table_hbm, out_hbm,
               table_v, in_v0, in_v1, out_v0, out_v1, tail_v,
               sem_in0, sem_in1, sem_out0, sem_out1):
    wid = lax.axis_index("s") * 2 + lax.axis_index("c")
    base = wid * CHUNK

    in_v = (in_v0, in_v1)
    out_v = (out_v0, out_v1)
    sem_in = (sem_in0, sem_in1)
    sem_out = (sem_out0, sem_out1)

    in_copies = [None] * NSUB
    out_copies = [None] * NSUB

    def start_in(s):
        b = s % 2
        in_copies[s] = pltpu.async_copy(
            z_hbm.at[pl.ds(base + s * SUB, SUB)], in_v[b], sem_in[b]
        )

    pltpu.sync_copy(table_hbm, table_v)
    start_in(0)

    for s in range(NSUB):
        b = s % 2
        if s + 1 < NSUB:
            start_in(s + 1)
        in_copies[s].wait()
        if s >= 2:
            out_copies[s - 2].wait()

        def body(i, carry, _b=b):
            off = i * STEP
            for j in range(UNROLL):
                sl = pl.ds(off + j * LANES, LANES)
                idx = in_v[_b][sl]
                out_v[_b][sl] = plsc.load_gather(table_v, [idx])
            return carry

        lax.fori_loop(0, SUB // STEP, body, 0)
        out_copies[s] = pltpu.async_copy(
            out_v[b], out_hbm.at[pl.ds(base + s * SUB, SUB)], sem_out[b]
        )

    out_copies[NSUB - 2].wait()
    out_copies[NSUB - 1].wait()

    @pl.when(wid == 0)
    def _():
        tail_base = NUM_WORKERS * CHUNK
        pltpu.sync_copy(z_hbm.at[pl.ds(tail_base, TAIL)], tail_v)

        def tbody(i, carry):
            off = i * STEP
            for j in range(UNROLL):
                sl = pl.ds(off + j * LANES, LANES)
                idx = tail_v[sl]
                tail_v[sl] = plsc.load_gather(table_v, [idx])
            return carry

        lax.fori_loop(0, TAIL // STEP, tbody, 0)
        pltpu.sync_copy(tail_v, out_hbm.at[pl.ds(tail_base, TAIL)])


def kernel(z, r, table):
    new_z = _translate(z, table)
    return (new_z, r * jnp.float32(1.0))


# speedup vs baseline: 1.1961x; 1.1961x over previous
import functools

import jax
import jax.numpy as jnp
from jax import lax
from jax.experimental import pallas as pl
from jax.experimental.pallas import tpu as pltpu
from jax.experimental.pallas import tpu_sc as plsc

N = 2_000_000
NUM_WORKERS = 32
LANES = 16
UNROLL = 8
STEP = LANES * UNROLL
NSUB = 8
SUB = 7_808
CHUNK = NSUB * SUB
TAIL = N - NUM_WORKERS * CHUNK
TABLE_LEN = 119

_mesh = plsc.VectorSubcoreMesh(core_axis_name="c", subcore_axis_name="s")


@functools.partial(
    pl.kernel,
    out_type=jax.ShapeDtypeStruct((N,), jnp.int32),
    mesh=_mesh,
    compiler_params=pltpu.CompilerParams(needs_layout_passes=False),
    scratch_types=[
        pltpu.VMEM((TABLE_LEN,), jnp.int32),
        pltpu.VMEM((SUB,), jnp.int32),
        pltpu.VMEM((SUB,), jnp.int32),
        pltpu.VMEM((SUB,), jnp.int32),
        pltpu.VMEM((SUB,), jnp.int32),
        pltpu.VMEM((TAIL,), jnp.int32),
        pltpu.SemaphoreType.DMA,
        pltpu.SemaphoreType.DMA,
        pltpu.SemaphoreType.DMA,
        pltpu.SemaphoreType.DMA,
    ],
)
def _translate(z_hbm, table_hbm, out_hbm,
               table_v, in_v0, in_v1, out_v0, out_v1, tail_v,
               sem_in0, sem_in1, sem_out0, sem_out1):
    wid = lax.axis_index("s") * 2 + lax.axis_index("c")
    base = wid * CHUNK

    in_v = (in_v0, in_v1)
    out_v = (out_v0, out_v1)
    sem_in = (sem_in0, sem_in1)
    sem_out = (sem_out0, sem_out1)

    in_copies = [None] * NSUB
    out_copies = [None] * NSUB

    def start_in(s):
        b = s % 2
        in_copies[s] = pltpu.async_copy(
            z_hbm.at[pl.ds(base + s * SUB, SUB)], in_v[b], sem_in[b]
        )

    pltpu.sync_copy(table_hbm, table_v)
    start_in(0)

    for s in range(NSUB):
        b = s % 2
        if s + 1 < NSUB:
            start_in(s + 1)
        in_copies[s].wait()
        if s >= 2:
            out_copies[s - 2].wait()

        iv, ov = in_v[b], out_v[b]

        @plsc.parallel_loop(0, SUB, step=LANES, unroll=UNROLL)
        def body(i, _iv=iv, _ov=ov):
            sl = pl.ds(i, LANES)
            _ov[sl] = plsc.load_gather(table_v, [_iv[sl]])
        out_copies[s] = pltpu.async_copy(
            out_v[b], out_hbm.at[pl.ds(base + s * SUB, SUB)], sem_out[b]
        )

    out_copies[NSUB - 2].wait()
    out_copies[NSUB - 1].wait()

    @pl.when(wid == 0)
    def _():
        tail_base = NUM_WORKERS * CHUNK
        pltpu.sync_copy(z_hbm.at[pl.ds(tail_base, TAIL)], tail_v)

        @plsc.parallel_loop(0, TAIL, step=LANES, unroll=UNROLL)
        def tbody(i):
            sl = pl.ds(i, LANES)
            tail_v[sl] = plsc.load_gather(table_v, [tail_v[sl]])
        pltpu.sync_copy(tail_v, out_hbm.at[pl.ds(tail_base, TAIL)])


def kernel(z, r, table):
    new_z = _translate(z, table)
    return (new_z, r * jnp.float32(1.0))


# speedup vs baseline: 1.2143x; 1.0151x over previous
import functools

import jax
import jax.numpy as jnp
from jax import lax
from jax.experimental import pallas as pl
from jax.experimental.pallas import tpu as pltpu
from jax.experimental.pallas import tpu_sc as plsc

N = 2_000_000
NUM_WORKERS = 32
LANES = 16
UNROLL = 8
STEP = LANES * UNROLL
NSUB = 8
SUB = 7_808
CHUNK = NSUB * SUB
TAIL = N - NUM_WORKERS * CHUNK
TABLE_LEN = 119

_mesh = plsc.VectorSubcoreMesh(core_axis_name="c", subcore_axis_name="s")


@functools.partial(
    pl.kernel,
    out_type=jax.ShapeDtypeStruct((N,), jnp.int32),
    mesh=_mesh,
    compiler_params=pltpu.CompilerParams(needs_layout_passes=False),
    scratch_types=[
        pltpu.VMEM((TABLE_LEN,), jnp.int32),
        pltpu.VMEM((SUB,), jnp.int32),
        pltpu.VMEM((SUB,), jnp.int32),
        pltpu.VMEM((SUB,), jnp.int32),
        pltpu.VMEM((SUB,), jnp.int32),
        pltpu.VMEM((TAIL,), jnp.int32),
        pltpu.SemaphoreType.DMA,
        pltpu.SemaphoreType.DMA,
        pltpu.SemaphoreType.DMA,
        pltpu.SemaphoreType.DMA,
    ],
)
def _translate(z_hbm, table_hbm, out_hbm,
               table_v, in_v0, in_v1, out_v0, out_v1, tail_v,
               sem_in0, sem_in1, sem_out0, sem_out1):
    wid = lax.axis_index("s") * 2 + lax.axis_index("c")
    base = wid * CHUNK

    bufs = ((in_v0, out_v0, sem_in0, sem_out0),
            (in_v1, out_v1, sem_in1, sem_out1))

    pltpu.sync_copy(table_hbm, table_v)
    pltpu.async_copy(z_hbm.at[pl.ds(base, SUB)], in_v0, sem_in0)
    pltpu.async_copy(z_hbm.at[pl.ds(base + SUB, SUB)], in_v1, sem_in1)

    def pair(g, carry):
        for half in range(2):
            iv, ov, s_in, s_out = bufs[half]
            off = base + (2 * g + half) * SUB
            pltpu.make_async_copy(z_hbm.at[pl.ds(off, SUB)], iv, s_in).wait()

            @pl.when(g >= 1)
            def _(ov=ov, s_out=s_out, off=off):
                pltpu.make_async_copy(
                    ov, out_hbm.at[pl.ds(off - 2 * SUB, SUB)], s_out
                ).wait()

            @plsc.parallel_loop(0, SUB, step=LANES, unroll=UNROLL)
            def body(i, _iv=iv, _ov=ov):
                sl = pl.ds(i, LANES)
                _ov[sl] = plsc.load_gather(table_v, [_iv[sl]])

            pltpu.async_copy(ov, out_hbm.at[pl.ds(off, SUB)], s_out)

            @pl.when(g < NSUB // 2 - 1)
            def _(iv=iv, s_in=s_in, off=off):
                pltpu.async_copy(
                    z_hbm.at[pl.ds(off + 2 * SUB, SUB)], iv, s_in
                )
        return carry

    lax.fori_loop(0, NSUB // 2, pair, 0)

    last0 = base + (NSUB - 2) * SUB
    pltpu.make_async_copy(
        out_v0, out_hbm.at[pl.ds(last0, SUB)], sem_out0).wait()
    pltpu.make_async_copy(
        out_v1, out_hbm.at[pl.ds(last0 + SUB, SUB)], sem_out1).wait()

    @pl.when(wid == 0)
    def _():
        tail_base = NUM_WORKERS * CHUNK
        pltpu.sync_copy(z_hbm.at[pl.ds(tail_base, TAIL)], tail_v)

        @plsc.parallel_loop(0, TAIL, step=LANES, unroll=UNROLL)
        def tbody(i):
            sl = pl.ds(i, LANES)
            tail_v[sl] = plsc.load_gather(table_v, [tail_v[sl]])
        pltpu.sync_copy(tail_v, out_hbm.at[pl.ds(tail_base, TAIL)])


def kernel(z, r, table):
    new_z = _translate(z, table)
    return (new_z, r)


# speedup vs baseline: 1.2177x; 1.0029x over previous
import functools

import jax
import jax.numpy as jnp
from jax import lax
from jax.experimental import pallas as pl
from jax.experimental.pallas import tpu as pltpu
from jax.experimental.pallas import tpu_sc as plsc

N = 2_000_000
NUM_WORKERS = 32
LANES = 16
UNROLL = 16
STEP = LANES * UNROLL
NSUB = 4
SUB = 15_616
CHUNK = NSUB * SUB
TAIL = N - NUM_WORKERS * CHUNK
TABLE_LEN = 119

_mesh = plsc.VectorSubcoreMesh(core_axis_name="c", subcore_axis_name="s")


@functools.partial(
    pl.kernel,
    out_type=jax.ShapeDtypeStruct((N,), jnp.int32),
    mesh=_mesh,
    compiler_params=pltpu.CompilerParams(needs_layout_passes=False),
    scratch_types=[
        pltpu.VMEM((TABLE_LEN,), jnp.int32),
        pltpu.VMEM((SUB,), jnp.int32),
        pltpu.VMEM((SUB,), jnp.int32),
        pltpu.VMEM((SUB,), jnp.int32),
        pltpu.VMEM((SUB,), jnp.int32),
        pltpu.VMEM((TAIL,), jnp.int32),
        pltpu.SemaphoreType.DMA,
        pltpu.SemaphoreType.DMA,
        pltpu.SemaphoreType.DMA,
        pltpu.SemaphoreType.DMA,
    ],
)
def _translate(z_hbm, table_hbm, out_hbm,
               table_v, in_v0, in_v1, out_v0, out_v1, tail_v,
               sem_in0, sem_in1, sem_out0, sem_out1):
    wid = lax.axis_index("s") * 2 + lax.axis_index("c")
    base = wid * CHUNK

    bufs = ((in_v0, out_v0, sem_in0, sem_out0),
            (in_v1, out_v1, sem_in1, sem_out1))

    pltpu.sync_copy(table_hbm, table_v)
    pltpu.async_copy(z_hbm.at[pl.ds(base, SUB)], in_v0, sem_in0)
    pltpu.async_copy(z_hbm.at[pl.ds(base + SUB, SUB)], in_v1, sem_in1)

    def pair(g, carry):
        for half in range(2):
            iv, ov, s_in, s_out = bufs[half]
            off = base + (2 * g + half) * SUB
            pltpu.make_async_copy(z_hbm.at[pl.ds(off, SUB)], iv, s_in).wait()

            @pl.when(g >= 1)
            def _(ov=ov, s_out=s_out, off=off):
                pltpu.make_async_copy(
                    ov, out_hbm.at[pl.ds(off - 2 * SUB, SUB)], s_out
                ).wait()

            @plsc.parallel_loop(0, SUB, step=LANES, unroll=UNROLL)
            def body(i, _iv=iv, _ov=ov):
                sl = pl.ds(i, LANES)
                _ov[sl] = plsc.load_gather(table_v, [_iv[sl]])

            pltpu.async_copy(ov, out_hbm.at[pl.ds(off, SUB)], s_out)

            @pl.when(g < NSUB // 2 - 1)
            def _(iv=iv, s_in=s_in, off=off):
                pltpu.async_copy(
                    z_hbm.at[pl.ds(off + 2 * SUB, SUB)], iv, s_in
                )
        return carry

    lax.fori_loop(0, NSUB // 2, pair, 0)

    last0 = base + (NSUB - 2) * SUB
    pltpu.make_async_copy(
        out_v0, out_hbm.at[pl.ds(last0, SUB)], sem_out0).wait()
    pltpu.make_async_copy(
        out_v1, out_hbm.at[pl.ds(last0 + SUB, SUB)], sem_out1).wait()

    @pl.when(wid == 0)
    def _():
        tail_base = NUM_WORKERS * CHUNK
        pltpu.sync_copy(z_hbm.at[pl.ds(tail_base, TAIL)], tail_v)

        @plsc.parallel_loop(0, TAIL, step=LANES, unroll=8)
        def tbody(i):
            sl = pl.ds(i, LANES)
            tail_v[sl] = plsc.load_gather(table_v, [tail_v[sl]])
        pltpu.sync_copy(tail_v, out_hbm.at[pl.ds(tail_base, TAIL)])


def kernel(z, r, table):
    new_z = _translate(z, table)
    return (new_z, r)


# speedup vs baseline: 1.2460x; 1.0232x over previous
import functools

import jax
import jax.numpy as jnp
from jax import lax
from jax.experimental import pallas as pl
from jax.experimental.pallas import tpu as pltpu
from jax.experimental.pallas import tpu_sc as plsc

N = 2_000_000
NUM_WORKERS = 32
LANES = 16
UNROLL = 16
STEP = LANES * UNROLL
NSUB = 4
SUB = 15_616
CHUNK = NSUB * SUB
TAIL = N - NUM_WORKERS * CHUNK
TABLE_LEN = 119

_mesh = plsc.VectorSubcoreMesh(core_axis_name="c", subcore_axis_name="s")


@functools.partial(
    pl.kernel,
    out_type=jax.ShapeDtypeStruct((N,), jnp.int32),
    mesh=_mesh,
    compiler_params=pltpu.CompilerParams(needs_layout_passes=False),
    scratch_types=[
        pltpu.VMEM((TABLE_LEN,), jnp.int32),
        pltpu.VMEM((SUB,), jnp.int32),
        pltpu.VMEM((SUB,), jnp.int32),
        pltpu.VMEM((SUB,), jnp.int32),
        pltpu.VMEM((SUB,), jnp.int32),
        pltpu.VMEM((TAIL,), jnp.int32),
        pltpu.SemaphoreType.DMA,
        pltpu.SemaphoreType.DMA,
        pltpu.SemaphoreType.DMA,
        pltpu.SemaphoreType.DMA,
    ],
)
def _translate(z_hbm, table_hbm, out_hbm,
               table_v, in_v0, in_v1, out_v0, out_v1, tail_v,
               sem_in0, sem_in1, sem_out0, sem_out1):
    wid = lax.axis_index("s") * 2 + lax.axis_index("c")
    base = wid * CHUNK

    bufs = ((in_v0, out_v0, sem_in0, sem_out0),
            (in_v1, out_v1, sem_in1, sem_out1))

    pltpu.async_copy(z_hbm.at[pl.ds(base, SUB)], in_v0, sem_in0)
    pltpu.async_copy(z_hbm.at[pl.ds(base + SUB, SUB)], in_v1, sem_in1)
    pltpu.sync_copy(table_hbm, table_v)

    def pair(g, carry):
        for half in range(2):
            iv, ov, s_in, s_out = bufs[half]
            off = base + (2 * g + half) * SUB
            pltpu.make_async_copy(z_hbm.at[pl.ds(off, SUB)], iv, s_in).wait()

            @pl.when(g >= 1)
            def _(ov=ov, s_out=s_out, off=off):
                pltpu.make_async_copy(
                    ov, out_hbm.at[pl.ds(off - 2 * SUB, SUB)], s_out
                ).wait()

            @plsc.parallel_loop(0, SUB, step=LANES, unroll=UNROLL)
            def body(i, _iv=iv, _ov=ov):
                sl = pl.ds(i, LANES)
                _ov[sl] = plsc.load_gather(table_v, [_iv[sl]])

            pltpu.async_copy(ov, out_hbm.at[pl.ds(off, SUB)], s_out)

            @pl.when(g < NSUB // 2 - 1)
            def _(iv=iv, s_in=s_in, off=off):
                pltpu.async_copy(
                    z_hbm.at[pl.ds(off + 2 * SUB, SUB)], iv, s_in
                )
        return carry

    lax.fori_loop(0, NSUB // 2, pair, 0)

    last0 = base + (NSUB - 2) * SUB
    pltpu.make_async_copy(
        out_v0, out_hbm.at[pl.ds(last0, SUB)], sem_out0).wait()
    pltpu.make_async_copy(
        out_v1, out_hbm.at[pl.ds(last0 + SUB, SUB)], sem_out1).wait()

    @pl.when(wid == 0)
    def _():
        tail_base = NUM_WORKERS * CHUNK
        pltpu.sync_copy(z_hbm.at[pl.ds(tail_base, TAIL)], tail_v)

        @plsc.parallel_loop(0, TAIL, step=LANES, unroll=8)
        def tbody(i):
            sl = pl.ds(i, LANES)
            tail_v[sl] = plsc.load_gather(table_v, [tail_v[sl]])
        pltpu.sync_copy(tail_v, out_hbm.at[pl.ds(tail_base, TAIL)])


def kernel(z, r, table):
    new_z = _translate(z, table)
    return (new_z, r)
